# baseline (speedup 1.0000x reference)
import functools

import jax
import jax.numpy as jnp
from jax import lax
from jax.experimental import pallas as pl
from jax.experimental.pallas import tpu as pltpu
from jax.experimental.pallas import tpu_sc as plsc

_NW = 32
_K = 32
_R = 32
_L = 16


def _make_sc_call(BE, BU, NF, NNZ_PAD, MAXBLK):
    OUT_ROWS = BU + 8
    NCH = NF // _L

    def body(feat_h, bc_h, br_h, g_h, occ_h, par_h, blk_h, out_h,
             occ_v, rows0, rows1, br0, br1, bc_v, g_v, val_v, rloc_v,
             blk_v, sidx_v, stage_v, par_v, sem0, sem1):
        wid = lax.axis_index("s") * 2 + lax.axis_index("c")

        pltpu.sync_copy(occ_h, occ_v)
        pltpu.sync_copy(par_h.at[wid], par_v)
        pvec = par_v[...]
        s = pvec[0]
        e = pvec[1]
        r0 = pvec[2]
        r1 = pvec[3]
        base_al = pvec[4]
        nb = pvec[5]
        nblk = pvec[6]

        pltpu.sync_copy(blk_h.at[wid], blk_v)

        iota = lax.iota(jnp.int32, _L)

        def blkend(k):
            kb = jnp.broadcast_to(k, (_L,))
            return plsc.load_gather(blk_v, [kb])[0]

        def zero_stage():
            @plsc.parallel_loop(0, _R)
            def _(rr):
                for c in range(NCH):
                    stage_v[rr, pl.ds(c * _L, _L)] = (
                        jnp.zeros((_L,), jnp.float32))

        def flush(k):
            row_base = r0 + k * _R
            for c in range(_R // _L):
                d = row_base + (c * _L) + iota
                sidx_v[pl.ds(c * _L, _L)] = jnp.where(
                    d < r1, d, jnp.int32(OUT_ROWS - 1))
            pltpu.sync_copy(stage_v, out_h.at[sidx_v])
            zero_stage()

        zero_stage()

        def issue(w, br_ref, rows_ref, sem):
            win_lo = pl.multiple_of(base_al + w * _K, 8)
            pltpu.sync_copy(br_h.at[pl.ds(win_lo, _K)], br_ref)
            pltpu.async_copy(feat_h.at[br_ref], rows_ref, sem)

        def compute(w, br_ref, rows_ref, sem, k):
            win_lo = pl.multiple_of(base_al + w * _K, 8)
            pltpu.sync_copy(bc_h.at[pl.ds(win_lo, _K)], bc_v)
            pltpu.sync_copy(g_h.at[pl.ds(win_lo, _K)], g_v)
            pltpu.make_async_copy(feat_h.at[br_ref], rows_ref, sem).wait()
            r0b = jnp.broadcast_to(r0, (_L,))
            for i in range(_K // _L):
                bcc = bc_v[pl.ds(i * _L, _L)]
                occ_b = plsc.load_gather(occ_v, [bcc])
                val_v[pl.ds(i * _L, _L)] = g_v[pl.ds(i * _L, _L)] / occ_b
                rloc_v[pl.ds(i * _L, _L)] = (bcc - r0b) & jnp.int32(_R - 1)

            jhi = jnp.minimum(e, win_lo + _K)

            def entries(jlo_, jhi_):
                @plsc.parallel_loop(jlo_, jhi_, unroll=2)
                def _(j):
                    jl = j - win_lo
                    jb = jnp.broadcast_to(jl, (_L,))
                    rl = plsc.load_gather(rloc_v, [jb])[0]
                    vb = plsc.load_gather(val_v, [jb])
                    for c in range(NCH):
                        x = rows_ref[jl, pl.ds(c * _L, _L)]
                        plsc.addupdate(
                            stage_v.at[rl, pl.ds(c * _L, _L)], vb * x)

            def wcond(kj):
                kk, jcur, jend = kj
                return jnp.logical_and(kk < nblk, jend <= jhi)

            def wbody(kj):
                kk, jcur, jend = kj
                entries(jcur, jend)
                flush(kk)
                return (kk + 1, jend, blkend(kk + 1))

            jcur0 = jnp.maximum(s, win_lo)
            k, jcur, _ = lax.while_loop(wcond, wbody, (k, jcur0, blkend(k)))
            entries(jcur, jhi)
            return k

        @pl.when(nb > 0)
        def _():
            issue(0, br0, rows0, sem0)

        def step(i, k):
            w0 = 2 * i
            w1 = w0 + 1

            @pl.when(w1 < nb)
            def _():
                issue(w1, br1, rows1, sem1)

            k = compute(w0, br0, rows0, sem0, k)

            @pl.when(w0 + 2 < nb)
            def _():
                issue(w0 + 2, br0, rows0, sem0)

            k = lax.cond(w1 < nb,
                         lambda kk: compute(w1, br1, rows1, sem1, kk),
                         lambda kk: kk, k)
            return k

        k = lax.fori_loop(0, (nb + 1) // 2, step, 0)

        def tcond(kk):
            return kk < nblk

        def tbody(kk):
            flush(kk)
            return kk + 1

        lax.while_loop(tcond, tbody, k)

    return pl.kernel(
        body,
        out_type=jax.ShapeDtypeStruct((OUT_ROWS, NF), jnp.float32),
        mesh=plsc.VectorSubcoreMesh(core_axis_name="c", subcore_axis_name="s",
                                    num_cores=2, num_subcores=16),
        compiler_params=pltpu.CompilerParams(
            needs_layout_passes=False,
            disable_bounds_checks=True,
            disable_semaphore_checks=True),
        scratch_types=[
            pltpu.VMEM((BU,), jnp.float32),
            pltpu.VMEM((_K, NF), jnp.float32),
            pltpu.VMEM((_K, NF), jnp.float32),
            pltpu.VMEM((_K,), jnp.int32),
            pltpu.VMEM((_K,), jnp.int32),
            pltpu.VMEM((_K,), jnp.int32),
            pltpu.VMEM((_K,), jnp.float32),
            pltpu.VMEM((_K,), jnp.float32),
            pltpu.VMEM((_K,), jnp.int32),
            pltpu.VMEM((MAXBLK,), jnp.int32),
            pltpu.VMEM((_R,), jnp.int32),
            pltpu.VMEM((_R, NF), jnp.float32),
            pltpu.VMEM((_L,), jnp.int32),
            pltpu.SemaphoreType.DMA,
            pltpu.SemaphoreType.DMA,
        ],
    )


def kernel(features, batch_idx, row_idx, col_idx, group_values, occurrences):
    B, NF, E = features.shape
    U = occurrences.shape[1]
    NNZ = batch_idx.shape[0]
    BU, BE = B * U, B * E

    bc = batch_idx * U + col_idx
    br = batch_idx * E + row_idx
    key = bc * (2 ** 15) + br
    key_s, g_s = lax.sort((key, group_values), num_keys=1)
    bc_s = key_s // (2 ** 15)
    br_s = key_s - bc_s * (2 ** 15)

    pad = 2 * _K
    bc_p = jnp.concatenate([bc_s, jnp.full((pad,), BU - 1, jnp.int32)])
    br_p = jnp.concatenate([br_s, jnp.zeros((pad,), jnp.int32)])
    g_p = jnp.concatenate([g_s, jnp.zeros((pad,), jnp.float32)])

    base = (jnp.arange(1, _NW) * NNZ) // _NW
    t_in = jnp.searchsorted(bc_s, bc_s[base], side="left").astype(jnp.int32)
    t_start = jnp.concatenate(
        [jnp.zeros((1,), jnp.int32), t_in, jnp.full((1,), NNZ, jnp.int32)])
    row_in = bc_p[t_in]
    row_start = jnp.concatenate(
        [jnp.zeros((1,), jnp.int32), row_in, jnp.full((1,), BU, jnp.int32)])

    s = t_start[:-1]
    e = t_start[1:]
    r0 = row_start[:-1]
    r1 = row_start[1:]
    base_al = s - (s % 16)
    nb = jnp.where(e > s, (e - base_al + _K - 1) // _K, 0)
    nblk = (r1 - r0 + _R - 1) // _R

    MAXBLK = BU // _R + 2
    MAXBLK += (-MAXBLK) % 8
    blk_rows = (r0[:, None] + (jnp.arange(MAXBLK)[None, :] + 1) * _R)
    blk_end = jnp.searchsorted(bc_s, blk_rows.reshape(-1),
                               side="left").astype(jnp.int32)
    blk_end = blk_end.reshape(_NW, MAXBLK)
    blk_end = jnp.clip(blk_end, s[:, None], e[:, None])

    params = jnp.zeros((_NW, _L), jnp.int32)
    params = params.at[:, 0].set(s).at[:, 1].set(e)
    params = params.at[:, 2].set(r0).at[:, 3].set(r1)
    params = params.at[:, 4].set(base_al).at[:, 5].set(nb)
    params = params.at[:, 6].set(nblk)

    feat_t = features.transpose(0, 2, 1).reshape(BE, NF)
    occ_flat = occurrences.reshape(BU)

    sc_call = _make_sc_call(BE, BU, NF, NNZ + pad, MAXBLK)
    out_t = sc_call(feat_t, bc_p, br_p, g_p, occ_flat, params, blk_end)

    return out_t[:BU].reshape(B, U, NF).transpose(0, 2, 1)


# speedup vs baseline: 3.4464x; 3.4464x over previous
import functools

import jax
import jax.numpy as jnp
from jax import lax
from jax.experimental import pallas as pl
from jax.experimental.pallas import tpu as pltpu
from jax.experimental.pallas import tpu_sc as plsc

_NW = 32
_K = 32
_R = 32
_L = 16


def _extract(vec, j):
    lane = lax.iota(jnp.int32, _L)
    return jnp.max(jnp.where(lane == j, vec, jnp.zeros_like(vec)))


def _make_sc_call(BE, BU, NF, NNZ_PAD):
    OUT_ROWS = BU + 8
    NCH = NF // _L

    def body(feat_h, bc_h, br_h, g_h, occ_h, par_h, out_h,
             occ_v, rows0, rows1, bc_v, br0, br1, g_v, val_v, sidx_v,
             stage_v, par_v, sem0, sem1):
        wid = lax.axis_index("s") * 2 + lax.axis_index("c")

        pltpu.sync_copy(par_h.at[wid], par_v)
        pvec = par_v[...]
        s = pvec[0]
        e = pvec[1]
        r0 = pvec[2]
        r1 = pvec[3]
        base_al = pvec[4]
        nb = pvec[5]

        pltpu.sync_copy(occ_h, occ_v)

        def zero_stage():
            def zr(r, carry):
                for c in range(NCH):
                    stage_v[r, pl.ds(c * _L, _L)] = jnp.zeros((_L,), jnp.float32)
                return carry
            lax.fori_loop(0, _R, zr, 0)

        def flush(row_base):
            for c in range(_R // _L):
                d = row_base + (c * _L) + lax.iota(jnp.int32, _L)
                sidx_v[pl.ds(c * _L, _L)] = jnp.where(
                    d < r1, d, jnp.int32(OUT_ROWS - 1))
            pltpu.sync_copy(stage_v, out_h.at[sidx_v])
            zero_stage()
            return row_base + _R

        zero_stage()

        def issue(w, br_ref, rows_ref, sem):
            win_lo = pl.multiple_of(base_al + w * _K, 8)
            pltpu.sync_copy(br_h.at[pl.ds(win_lo, _K)], br_ref)
            pltpu.async_copy(feat_h.at[br_ref], rows_ref, sem)

        def compute(w, br_ref, rows_ref, sem, row_base):
            win_lo = pl.multiple_of(base_al + w * _K, 8)
            pltpu.sync_copy(bc_h.at[pl.ds(win_lo, _K)], bc_v)
            pltpu.sync_copy(g_h.at[pl.ds(win_lo, _K)], g_v)
            pltpu.make_async_copy(feat_h.at[br_ref], rows_ref, sem).wait()
            for c in range(_K // _L):
                bcc = bc_v[pl.ds(c * _L, _L)]
                occ_b = plsc.load_gather(occ_v, [bcc])
                val_v[pl.ds(c * _L, _L)] = g_v[pl.ds(c * _L, _L)] / occ_b

            jlo = jnp.maximum(s, win_lo) - win_lo
            jhi = jnp.minimum(e, win_lo + _K) - win_lo

            def entry(j, rb):
                jb = jnp.broadcast_to(j, (_L,))
                bcj = plsc.load_gather(bc_v, [jb])[0]
                rb = lax.while_loop(lambda r: bcj >= r + _R, flush, rb)
                r_loc = bcj - rb
                vb = plsc.load_gather(val_v, [jb])
                for c in range(NCH):
                    plsc.addupdate(
                        stage_v.at[r_loc, pl.ds(c * _L, _L)],
                        vb * rows_ref[j, pl.ds(c * _L, _L)])
                return rb

            return lax.fori_loop(jlo, jhi, entry, row_base)

        @pl.when(nb > 0)
        def _():
            issue(0, br0, rows0, sem0)

        def step(i, row_base):
            w0 = 2 * i
            w1 = w0 + 1

            @pl.when(w1 < nb)
            def _():
                issue(w1, br1, rows1, sem1)

            row_base = compute(w0, br0, rows0, sem0, row_base)

            @pl.when(w0 + 2 < nb)
            def _():
                issue(w0 + 2, br0, rows0, sem0)

            row_base = lax.cond(
                w1 < nb,
                lambda rb: compute(w1, br1, rows1, sem1, rb),
                lambda rb: rb, row_base)
            return row_base

        row_base = lax.fori_loop(0, (nb + 1) // 2, step, r0)
        lax.while_loop(lambda r: r < r1, flush, row_base)

    return pl.kernel(
        body,
        out_type=jax.ShapeDtypeStruct((OUT_ROWS, NF), jnp.float32),
        mesh=plsc.VectorSubcoreMesh(core_axis_name="c", subcore_axis_name="s",
                                    num_cores=2, num_subcores=16),
        compiler_params=pltpu.CompilerParams(
            needs_layout_passes=False,
            disable_bounds_checks=True,
            disable_semaphore_checks=True),
        scratch_types=[
            pltpu.VMEM((BU,), jnp.float32),
            pltpu.VMEM((_K, NF), jnp.float32),
            pltpu.VMEM((_K, NF), jnp.float32),
            pltpu.VMEM((_K,), jnp.int32),
            pltpu.VMEM((_K,), jnp.int32),
            pltpu.VMEM((_K,), jnp.int32),
            pltpu.VMEM((_K,), jnp.float32),
            pltpu.VMEM((_K,), jnp.float32),
            pltpu.VMEM((_R,), jnp.int32),
            pltpu.VMEM((_R, NF), jnp.float32),
            pltpu.VMEM((_L,), jnp.int32),
            pltpu.SemaphoreType.DMA,
            pltpu.SemaphoreType.DMA,
        ],
    )


def kernel(features, batch_idx, row_idx, col_idx, group_values, occurrences):
    B, NF, E = features.shape
    U = occurrences.shape[1]
    NNZ = batch_idx.shape[0]
    BU, BE = B * U, B * E

    bc = batch_idx * U + col_idx
    br = batch_idx * E + row_idx
    key = bc * (2 ** 15) + br
    key_s, g_s = lax.sort((key, group_values), num_keys=1)
    bc_s = key_s // (2 ** 15)
    br_s = key_s - bc_s * (2 ** 15)

    pad = 2 * _K
    bc_p = jnp.concatenate([bc_s, jnp.full((pad,), BU - 1, jnp.int32)])
    br_p = jnp.concatenate([br_s, jnp.zeros((pad,), jnp.int32)])
    g_p = jnp.concatenate([g_s, jnp.zeros((pad,), jnp.float32)])

    base = (jnp.arange(1, _NW) * NNZ) // _NW
    t_in = jnp.searchsorted(bc_s, bc_s[base], side="left").astype(jnp.int32)
    t_start = jnp.concatenate(
        [jnp.zeros((1,), jnp.int32), t_in, jnp.full((1,), NNZ, jnp.int32)])
    row_in = bc_p[t_in]
    row_start = jnp.concatenate(
        [jnp.zeros((1,), jnp.int32), row_in, jnp.full((1,), BU, jnp.int32)])

    s = t_start[:-1]
    e = t_start[1:]
    base_al = s - (s % 8)
    nb = jnp.where(e > s, (e - base_al + _K - 1) // _K, 0)
    params = jnp.zeros((_NW, _L), jnp.int32)
    params = params.at[:, 0].set(s).at[:, 1].set(e)
    params = params.at[:, 2].set(row_start[:-1]).at[:, 3].set(row_start[1:])
    params = params.at[:, 4].set(base_al).at[:, 5].set(nb)

    feat_t = features.transpose(0, 2, 1).reshape(BE, NF)
    occ_flat = occurrences.reshape(BU)

    sc_call = _make_sc_call(BE, BU, NF, NNZ + pad)
    out_t = sc_call(feat_t, bc_p, br_p, g_p, occ_flat, params)

    return out_t[:BU].reshape(B, U, NF).transpose(0, 2, 1)
